# baseline (speedup 1.0000x reference)
import functools

import jax
import jax.numpy as jnp
from jax import lax
from jax.experimental import pallas as pl
from jax.experimental.pallas import tpu as pltpu
from jax.experimental.pallas import tpu_sc as plsc

N_NODES = 10000
N_PAD = 10240
E = 320000
NC, NS = 2, 16
NW = NC * NS
E_W = 10240
E_PAD = NW * E_W
C = 128
NCH = E_W // C
ROWS_T = N_PAD // NS
W_DEG = 8
WIDE = 128

_mesh = lambda: plsc.VectorSubcoreMesh(
    core_axis_name="c", subcore_axis_name="s", num_cores=NC, num_subcores=NS)

_sc_params = pltpu.CompilerParams(use_tc_tiling_on_sc=False)


def _sc_degree(dst_g, zeros_deg, ones_rows):
  @functools.partial(
      pl.kernel,
      out_type=jax.ShapeDtypeStruct((NC, N_PAD, W_DEG), jnp.float32),
      mesh=_mesh(),
      compiler_params=_sc_params,
      scratch_types=[
          pltpu.VMEM((NCH, C), jnp.int32),
          pltpu.VMEM((C, W_DEG), jnp.float32),
          pltpu.VMEM_SHARED((N_PAD, W_DEG), jnp.float32),
      ],
  )
  def k(dst_hbm, zeros_hbm, ones_hbm, out_hbm, dst_v, ones_v, acc):
    c = lax.axis_index("c")
    s = lax.axis_index("s")
    w = s * NC + c
    pltpu.sync_copy(dst_hbm.at[w], dst_v)
    pltpu.sync_copy(ones_hbm, ones_v)
    sl = pl.ds(s * ROWS_T, ROWS_T)
    pltpu.sync_copy(zeros_hbm.at[sl], acc.at[sl])
    plsc.subcore_barrier()

    def step(j, carry):
      pltpu.sync_copy(ones_v, acc.at[dst_v.at[j]], add=True)
      return carry

    lax.fori_loop(0, NCH, step, 0)
    plsc.subcore_barrier()
    pltpu.sync_copy(acc.at[sl], out_hbm.at[c, sl])

  return k(dst_g, zeros_deg, ones_rows)


def _make_sc_propagate(Dg):
  @functools.partial(
      pl.kernel,
      out_type=jax.ShapeDtypeStruct((NC, N_PAD, WIDE), jnp.float32),
      mesh=_mesh(),
      compiler_params=_sc_params,
      scratch_types=[
          pltpu.VMEM((NCH, C), jnp.int32),
          pltpu.VMEM((NCH, C), jnp.int32),
          [pltpu.VMEM((C, Dg), jnp.float32) for _ in range(4)],
          pltpu.VMEM_SHARED((N_PAD, Dg), jnp.float32),
          [pltpu.SemaphoreType.DMA for _ in range(4)],
          [pltpu.SemaphoreType.DMA for _ in range(4)],
      ],
  )
  def k(h_hbm, src_hbm, dst_hbm, zeros_hbm, out_hbm,
        src_v, dst_v, bufs, acc, gsems, ssems):
    c = lax.axis_index("c")
    s = lax.axis_index("s")
    w = s * NC + c
    pltpu.sync_copy(src_hbm.at[w], src_v)
    pltpu.sync_copy(dst_hbm.at[w], dst_v)
    sl = pl.ds(s * ROWS_T, ROWS_T)
    pltpu.sync_copy(zeros_hbm.at[sl], acc.at[sl])
    plsc.subcore_barrier()

    def gather(j, b):
      pltpu.async_copy(h_hbm.at[src_v.at[j]], bufs[b], gsems[b])

    gather(0, 0)
    gather(1, 1)

    def step(i, carry):
      for u in range(4):
        kk = 4 * i + u
        pltpu.make_async_copy(h_hbm.at[src_v.at[kk]], bufs[u], gsems[u]).wait()
        pltpu.async_copy(bufs[u], acc.at[dst_v.at[kk]], ssems[u], add=True)

        @pl.when(kk + 2 < NCH)
        def _():
          @pl.when(kk >= 2)
          def _():
            pltpu.make_async_copy(
                bufs[(u + 2) % 4], acc.at[dst_v.at[kk - 2]],
                ssems[(u + 2) % 4]).wait()
          gather(kk + 2, (u + 2) % 4)

      return carry

    lax.fori_loop(0, NCH // 4, step, 0)
    for j in (NCH - 4, NCH - 3, NCH - 2, NCH - 1):
      pltpu.make_async_copy(
          bufs[j % 4], acc.at[dst_v.at[j]], ssems[j % 4]).wait()
    plsc.subcore_barrier()
    pltpu.sync_copy(acc.at[sl], out_hbm.at[c, sl, pl.ds(0, Dg)])

  return k


def _tc_prep(x_pad, W1, dis):
  d1 = W1.shape[1]

  def body(x_ref, w_ref, dis_ref, h_ref):
    h_ref[:, :d1] = jnp.dot(x_ref[...], w_ref[...],
                            preferred_element_type=jnp.float32) * dis_ref[...]
    h_ref[:, d1:] = jnp.zeros((N_PAD, WIDE - d1), jnp.float32)

  return pl.pallas_call(
      body,
      out_shape=jax.ShapeDtypeStruct((N_PAD, WIDE), jnp.float32),
  )(x_pad, W1, dis)


def _tc_combine_matmul(p, hp_w, dis, b2d, W, d_in):
  dout = W.shape[1]

  def body(p_ref, hp_ref, dis_ref, b_ref, w_ref, o_ref):
    dis = dis_ref[...]
    z = dis * (p_ref[0, :, :d_in] + p_ref[1, :, :d_in]
               + hp_ref[:, :d_in]) + b_ref[...]
    z = jnp.maximum(z, 0.0)
    o_ref[:, :dout] = jnp.dot(z, w_ref[...],
                              preferred_element_type=jnp.float32) * dis
    o_ref[:, dout:] = jnp.zeros((N_PAD, WIDE - dout), jnp.float32)

  return pl.pallas_call(
      body,
      out_shape=jax.ShapeDtypeStruct((N_PAD, WIDE), jnp.float32),
  )(p, hp_w, dis, b2d, W)


def _tc_final(p, hp_w, dis, b2d, dout):
  def body(p_ref, hp_ref, dis_ref, b_ref, o_ref):
    z = (dis_ref[:N_NODES] * (p_ref[0, :N_NODES, :dout]
                              + p_ref[1, :N_NODES, :dout]
                              + hp_ref[:N_NODES, :dout]) + b_ref[...])
    m = jnp.max(z, axis=1, keepdims=True)
    e = jnp.exp(z - m)
    lse = jnp.log(jnp.sum(e, axis=1, keepdims=True)) + m
    o_ref[...] = z - lse

  return pl.pallas_call(
      body,
      out_shape=jax.ShapeDtypeStruct((N_NODES, dout), jnp.float32),
  )(p, hp_w, dis, b2d)


def kernel(x, edge_index, W1, b1, W2, b2, W3, b3):
  src = edge_index[0].astype(jnp.int32)
  dst = edge_index[1].astype(jnp.int32)
  n_pad_edges = E_PAD - E
  pad_idx = (N_NODES
             + jnp.arange(n_pad_edges, dtype=jnp.int32) % (N_PAD - N_NODES))
  src_p = jnp.concatenate([src, pad_idx])
  dst_g = jnp.concatenate([dst, pad_idx]).reshape(NW, NCH, C)
  src_g2 = (src_p * 2).reshape(NW, NCH, C)
  src_g4 = (src_p * 4).reshape(NW, NCH, C)

  x_pad = jnp.pad(x, ((0, N_PAD - N_NODES), (0, 0)))
  d1, d2, d3 = W1.shape[1], W2.shape[1], W3.shape[1]

  zeros_deg = jnp.zeros((N_PAD, W_DEG), jnp.float32)
  ones_rows = jnp.ones((C, W_DEG), jnp.float32)
  deg_p = _sc_degree(dst_g, zeros_deg, ones_rows)

  dis = lax.rsqrt(deg_p[0, :, 0] + deg_p[1, :, 0] + 1.0)[:, None]
  h1w = _tc_prep(x_pad, W1, dis)
  zeros64 = jnp.zeros((N_PAD, 64), jnp.float32)
  zeros32 = jnp.zeros((N_PAD, 32), jnp.float32)

  prop64 = _make_sc_propagate(64)
  prop32 = _make_sc_propagate(32)

  p1 = prop64(h1w.reshape(2 * N_PAD, 64), src_g2, dst_g, zeros64)
  h2w = _tc_combine_matmul(p1, h1w, dis, b1.reshape(1, d1), W2, d1)

  p2 = prop32(h2w.reshape(4 * N_PAD, 32), src_g4, dst_g, zeros32)
  h3w = _tc_combine_matmul(p2, h2w, dis, b2.reshape(1, d2), W3, d2)

  p3 = prop64(h3w.reshape(2 * N_PAD, 64), src_g2, dst_g, zeros64)
  return _tc_final(p3, h3w, dis, b3.reshape(1, d3), d3)


# speedup vs baseline: 1.0087x; 1.0087x over previous
import functools

import jax
import jax.numpy as jnp
from jax import lax
from jax.experimental import pallas as pl
from jax.experimental.pallas import tpu as pltpu
from jax.experimental.pallas import tpu_sc as plsc

N_NODES = 10000
N_PAD = 10240
E = 320000
NC, NS = 2, 16
NW = NC * NS
E_W = 10240
E_PAD = NW * E_W
C = 128
NCH = E_W // C
ROWS_T = N_PAD // NS
W_DEG = 8
WIDE = 128

_mesh = lambda: plsc.VectorSubcoreMesh(
    core_axis_name="c", subcore_axis_name="s", num_cores=NC, num_subcores=NS)

_sc_params = pltpu.CompilerParams(use_tc_tiling_on_sc=False)


def _sc_degree(dst_g, zeros_deg, ones_rows):
  @functools.partial(
      pl.kernel,
      out_type=jax.ShapeDtypeStruct((NC, N_PAD, WIDE), jnp.float32),
      mesh=_mesh(),
      compiler_params=_sc_params,
      scratch_types=[
          pltpu.VMEM((NCH, C), jnp.int32),
          pltpu.VMEM((C, W_DEG), jnp.float32),
          pltpu.VMEM_SHARED((N_PAD, W_DEG), jnp.float32),
      ],
  )
  def k(dst_hbm, zeros_hbm, ones_hbm, out_hbm, dst_v, ones_v, acc):
    c = lax.axis_index("c")
    s = lax.axis_index("s")
    w = s * NC + c
    pltpu.sync_copy(dst_hbm.at[w], dst_v)
    pltpu.sync_copy(ones_hbm, ones_v)
    sl = pl.ds(s * ROWS_T, ROWS_T)
    pltpu.sync_copy(zeros_hbm.at[sl], acc.at[sl])
    plsc.subcore_barrier()

    def step(j, carry):
      pltpu.sync_copy(ones_v, acc.at[dst_v.at[j]], add=True)
      return carry

    lax.fori_loop(0, NCH, step, 0)
    plsc.subcore_barrier()
    pltpu.sync_copy(acc.at[sl], out_hbm.at[c, sl, pl.ds(0, W_DEG)])

  return k(dst_g, zeros_deg, ones_rows)


def _make_sc_propagate(Dg):
  @functools.partial(
      pl.kernel,
      out_type=jax.ShapeDtypeStruct((NC, N_PAD, WIDE), jnp.float32),
      mesh=_mesh(),
      compiler_params=_sc_params,
      scratch_types=[
          pltpu.VMEM((NCH, C), jnp.int32),
          pltpu.VMEM((NCH, C), jnp.int32),
          [pltpu.VMEM((C, Dg), jnp.float32) for _ in range(4)],
          pltpu.VMEM_SHARED((N_PAD, Dg), jnp.float32),
          [pltpu.SemaphoreType.DMA for _ in range(4)],
          [pltpu.SemaphoreType.DMA for _ in range(4)],
      ],
  )
  def k(h_hbm, src_hbm, dst_hbm, zeros_hbm, out_hbm,
        src_v, dst_v, bufs, acc, gsems, ssems):
    c = lax.axis_index("c")
    s = lax.axis_index("s")
    w = s * NC + c
    pltpu.sync_copy(src_hbm.at[w], src_v)
    pltpu.sync_copy(dst_hbm.at[w], dst_v)
    sl = pl.ds(s * ROWS_T, ROWS_T)
    pltpu.sync_copy(zeros_hbm.at[sl], acc.at[sl])
    plsc.subcore_barrier()

    def gather(j, b):
      pltpu.async_copy(h_hbm.at[src_v.at[j]], bufs[b], gsems[b])

    gather(0, 0)
    gather(1, 1)

    def step(i, carry):
      for u in range(4):
        kk = 4 * i + u
        pltpu.make_async_copy(h_hbm.at[src_v.at[kk]], bufs[u], gsems[u]).wait()
        pltpu.async_copy(bufs[u], acc.at[dst_v.at[kk]], ssems[u], add=True)

        @pl.when(kk + 2 < NCH)
        def _():
          @pl.when(kk >= 2)
          def _():
            pltpu.make_async_copy(
                bufs[(u + 2) % 4], acc.at[dst_v.at[kk - 2]],
                ssems[(u + 2) % 4]).wait()
          gather(kk + 2, (u + 2) % 4)

      return carry

    lax.fori_loop(0, NCH // 4, step, 0)
    for j in (NCH - 4, NCH - 3, NCH - 2, NCH - 1):
      pltpu.make_async_copy(
          bufs[j % 4], acc.at[dst_v.at[j]], ssems[j % 4]).wait()
    plsc.subcore_barrier()
    pltpu.sync_copy(acc.at[sl], out_hbm.at[c, sl, pl.ds(0, Dg)])

  return k


def _tc_prep(x_pad, W1):
  d1 = W1.shape[1]

  def body(x_ref, w_ref, h_ref):
    h_ref[:, :d1] = jnp.dot(x_ref[...], w_ref[...],
                            preferred_element_type=jnp.float32)
    h_ref[:, d1:] = jnp.zeros((N_PAD, WIDE - d1), jnp.float32)

  return pl.pallas_call(
      body,
      out_shape=jax.ShapeDtypeStruct((N_PAD, WIDE), jnp.float32),
  )(x_pad, W1)


def _tc_combine_matmul(p, hp_w, dis, b2d, W, d_in):
  dout = W.shape[1]

  def body(p_ref, hp_ref, dis_ref, b_ref, w_ref, o_ref):
    dis = dis_ref[...]
    z = dis * (p_ref[0, :, :d_in] + p_ref[1, :, :d_in]
               + hp_ref[:, :d_in]) + b_ref[...]
    z = jnp.maximum(z, 0.0)
    o_ref[:, :dout] = jnp.dot(z, w_ref[...],
                              preferred_element_type=jnp.float32) * dis
    o_ref[:, dout:] = jnp.zeros((N_PAD, WIDE - dout), jnp.float32)

  return pl.pallas_call(
      body,
      out_shape=jax.ShapeDtypeStruct((N_PAD, WIDE), jnp.float32),
  )(p, hp_w, dis, b2d, W)


def _tc_final(p, hp_w, dis, b2d, dout):
  def body(p_ref, hp_ref, dis_ref, b_ref, o_ref):
    z = (dis_ref[...] * (p_ref[0, :, :dout] + p_ref[1, :, :dout]
                         + hp_ref[:, :dout]) + b_ref[...])
    m = jnp.max(z, axis=1, keepdims=True)
    e = jnp.exp(z - m)
    lse = jnp.log(jnp.sum(e, axis=1, keepdims=True)) + m
    o_ref[...] = z - lse

  return pl.pallas_call(
      body,
      out_shape=jax.ShapeDtypeStruct((N_PAD, dout), jnp.float32),
  )(p, hp_w, dis, b2d)


def kernel(x, edge_index, W1, b1, W2, b2, W3, b3):
  eit = edge_index.astype(jnp.int32).reshape(2, E // C, C)
  n_pad_edges = E_PAD - E
  pad_idx = (N_NODES
             + jnp.arange(n_pad_edges, dtype=jnp.int32) % (N_PAD - N_NODES))
  pad_blk = pad_idx.reshape(n_pad_edges // C, C)
  src_c = jnp.concatenate([eit[0], pad_blk]).reshape(NW, NCH, C)
  dst_g = jnp.concatenate([eit[1], pad_blk]).reshape(NW, NCH, C)
  src_g2 = src_c * 2
  src_g4 = src_c * 4

  x_pad = jnp.pad(x, ((0, N_PAD - N_NODES), (0, 0)))
  d1, d2, d3 = W1.shape[1], W2.shape[1], W3.shape[1]

  zeros_deg = jnp.zeros((N_PAD, W_DEG), jnp.float32)
  ones_rows = jnp.ones((C, W_DEG), jnp.float32)
  deg_p = _sc_degree(dst_g, zeros_deg, ones_rows)
  m1w = _tc_prep(x_pad, W1)

  dis = lax.rsqrt(deg_p[0, :, 0] + deg_p[1, :, 0] + 1.0)[:, None]
  h1w = m1w * dis
  zeros64 = jnp.zeros((N_PAD, 64), jnp.float32)
  zeros32 = jnp.zeros((N_PAD, 32), jnp.float32)

  prop64 = _make_sc_propagate(64)
  prop32 = _make_sc_propagate(32)

  p1 = prop64(h1w.reshape(2 * N_PAD, 64), src_g2, dst_g, zeros64)
  h2w = _tc_combine_matmul(p1, h1w, dis, b1.reshape(1, d1), W2, d1)

  p2 = prop32(h2w.reshape(4 * N_PAD, 32), src_g4, dst_g, zeros32)
  h3w = _tc_combine_matmul(p2, h2w, dis, b2.reshape(1, d2), W3, d2)

  p3 = prop64(h3w.reshape(2 * N_PAD, 64), src_g2, dst_g, zeros64)
  return _tc_final(p3, h3w, dis, b3.reshape(1, d3), d3)[:N_NODES]


# speedup vs baseline: 1.0434x; 1.0344x over previous
import functools

import jax
import jax.numpy as jnp
from jax import lax
from jax.experimental import pallas as pl
from jax.experimental.pallas import tpu as pltpu
from jax.experimental.pallas import tpu_sc as plsc

N_NODES = 10000
N_PAD = 10240
E = 320000
NC, NS = 2, 16
NW = NC * NS
E_W = 10240
E_PAD = NW * E_W
C = 128
NCH = E_W // C
ROWS_T = N_PAD // NS
W_DEG = 8
WIDE = 128

_mesh = lambda: plsc.VectorSubcoreMesh(
    core_axis_name="c", subcore_axis_name="s", num_cores=NC, num_subcores=NS)

_sc_params = pltpu.CompilerParams(use_tc_tiling_on_sc=False)
_sc_params_nl = pltpu.CompilerParams(
    use_tc_tiling_on_sc=False, needs_layout_passes=False)


def _sc_degree(dst_g, zeros_deg, ones_rows):
  @functools.partial(
      pl.kernel,
      out_type=jax.ShapeDtypeStruct((NC, N_PAD // C, C), jnp.float32),
      mesh=_mesh(),
      compiler_params=_sc_params_nl,
      scratch_types=[
          pltpu.VMEM((NCH, C), jnp.int32),
          pltpu.VMEM((C, W_DEG), jnp.float32),
          pltpu.VMEM((ROWS_T, W_DEG), jnp.float32),
          pltpu.VMEM((ROWS_T // C, C), jnp.float32),
          pltpu.VMEM_SHARED((N_PAD, W_DEG), jnp.float32),
      ],
  )
  def k(dst_hbm, zeros_hbm, ones_hbm, out_hbm, dst_v, ones_v, cin, cout, acc):
    c = lax.axis_index("c")
    s = lax.axis_index("s")
    w = s * NC + c
    pltpu.sync_copy(dst_hbm.at[w], dst_v)
    pltpu.sync_copy(ones_hbm, ones_v)
    sl = pl.ds(s * ROWS_T, ROWS_T)
    pltpu.sync_copy(zeros_hbm.at[sl], acc.at[sl])
    plsc.subcore_barrier()

    def step(j, carry):
      pltpu.sync_copy(ones_v, acc.at[dst_v.at[j]], add=True)
      return carry

    lax.fori_loop(0, NCH, step, 0)
    plsc.subcore_barrier()
    pltpu.sync_copy(acc.at[sl], cin)
    iota16 = lax.iota(jnp.int32, 16)
    zeros16 = jnp.zeros((16,), jnp.int32)

    for t in range(ROWS_T // 16):
      vals = plsc.load_gather(cin, [t * 16 + iota16, zeros16])
      cout[t // 8, pl.ds((t % 8) * 16, 16)] = vals
    pltpu.sync_copy(cout, out_hbm.at[c, pl.ds(s * (ROWS_T // C), ROWS_T // C)])

  return k(dst_g, zeros_deg, ones_rows)


def _make_sc_propagate(Dg):
  @functools.partial(
      pl.kernel,
      out_type=jax.ShapeDtypeStruct((NC, N_PAD, WIDE), jnp.float32),
      mesh=_mesh(),
      compiler_params=_sc_params,
      scratch_types=[
          pltpu.VMEM((NCH, C), jnp.int32),
          pltpu.VMEM((NCH, C), jnp.int32),
          [pltpu.VMEM((C, Dg), jnp.float32) for _ in range(4)],
          pltpu.VMEM_SHARED((N_PAD, Dg), jnp.float32),
          [pltpu.SemaphoreType.DMA for _ in range(4)],
          [pltpu.SemaphoreType.DMA for _ in range(4)],
      ],
  )
  def k(h_hbm, src_hbm, dst_hbm, zeros_hbm, out_hbm,
        src_v, dst_v, bufs, acc, gsems, ssems):
    c = lax.axis_index("c")
    s = lax.axis_index("s")
    w = s * NC + c
    pltpu.sync_copy(src_hbm.at[w], src_v)
    pltpu.sync_copy(dst_hbm.at[w], dst_v)
    sl = pl.ds(s * ROWS_T, ROWS_T)
    pltpu.sync_copy(zeros_hbm.at[sl], acc.at[sl])
    plsc.subcore_barrier()

    def gather(j, b):
      pltpu.async_copy(h_hbm.at[src_v.at[j]], bufs[b], gsems[b])

    gather(0, 0)
    gather(1, 1)

    def step(i, carry):
      for u in range(4):
        kk = 4 * i + u
        pltpu.make_async_copy(h_hbm.at[src_v.at[kk]], bufs[u], gsems[u]).wait()
        pltpu.async_copy(bufs[u], acc.at[dst_v.at[kk]], ssems[u], add=True)

        @pl.when(kk + 2 < NCH)
        def _():
          @pl.when(kk >= 2)
          def _():
            pltpu.make_async_copy(
                bufs[(u + 2) % 4], acc.at[dst_v.at[kk - 2]],
                ssems[(u + 2) % 4]).wait()
          gather(kk + 2, (u + 2) % 4)

      return carry

    lax.fori_loop(0, NCH // 4, step, 0)
    for j in (NCH - 4, NCH - 3, NCH - 2, NCH - 1):
      pltpu.make_async_copy(
          bufs[j % 4], acc.at[dst_v.at[j]], ssems[j % 4]).wait()
    plsc.subcore_barrier()
    pltpu.sync_copy(acc.at[sl], out_hbm.at[c, sl, pl.ds(0, Dg)])

  return k


def _tc_prep(x_pad, W1):
  d1 = W1.shape[1]

  def body(x_ref, w_ref, h_ref):
    h_ref[:, :d1] = jnp.dot(x_ref[...], w_ref[...],
                            preferred_element_type=jnp.float32)
    h_ref[:, d1:] = jnp.zeros((N_PAD, WIDE - d1), jnp.float32)

  return pl.pallas_call(
      body,
      out_shape=jax.ShapeDtypeStruct((N_PAD, WIDE), jnp.float32),
  )(x_pad, W1)


def _tc_combine_matmul(p, hp_w, dis, b2d, W, d_in):
  dout = W.shape[1]

  def body(p_ref, hp_ref, dis_ref, b_ref, w_ref, o_ref):
    dis = dis_ref[...]
    z = dis * (p_ref[0, :, :d_in] + p_ref[1, :, :d_in]
               + hp_ref[:, :d_in]) + b_ref[...]
    z = jnp.maximum(z, 0.0)
    o_ref[:, :dout] = jnp.dot(z, w_ref[...],
                              preferred_element_type=jnp.float32) * dis
    o_ref[:, dout:] = jnp.zeros((N_PAD, WIDE - dout), jnp.float32)

  return pl.pallas_call(
      body,
      out_shape=jax.ShapeDtypeStruct((N_PAD, WIDE), jnp.float32),
  )(p, hp_w, dis, b2d, W)


def _tc_final(p, hp_w, dis, b2d, dout):
  def body(p_ref, hp_ref, dis_ref, b_ref, o_ref):
    z = (dis_ref[...] * (p_ref[0, :, :dout] + p_ref[1, :, :dout]
                         + hp_ref[:, :dout]) + b_ref[...])
    m = jnp.max(z, axis=1, keepdims=True)
    e = jnp.exp(z - m)
    lse = jnp.log(jnp.sum(e, axis=1, keepdims=True)) + m
    o_ref[...] = z - lse

  return pl.pallas_call(
      body,
      out_shape=jax.ShapeDtypeStruct((N_PAD, dout), jnp.float32),
  )(p, hp_w, dis, b2d)


def kernel(x, edge_index, W1, b1, W2, b2, W3, b3):
  src = edge_index[0].astype(jnp.int32)
  dst = edge_index[1].astype(jnp.int32)
  n_pad_edges = E_PAD - E
  pad_idx = (N_NODES
             + jnp.arange(n_pad_edges, dtype=jnp.int32) % (N_PAD - N_NODES))
  src_p = jnp.concatenate([src, pad_idx])
  dst_g = jnp.concatenate([dst, pad_idx]).reshape(NW, NCH, C)
  src_g2 = (src_p * 2).reshape(NW, NCH, C)
  src_g4 = (src_p * 4).reshape(NW, NCH, C)

  x_pad = jnp.pad(x, ((0, N_PAD - N_NODES), (0, 0)))
  d1, d2, d3 = W1.shape[1], W2.shape[1], W3.shape[1]

  zeros_deg = jnp.zeros((N_PAD, W_DEG), jnp.float32)
  ones_rows = jnp.ones((C, W_DEG), jnp.float32)
  deg_c = _sc_degree(dst_g, zeros_deg, ones_rows)
  m1w = _tc_prep(x_pad, W1)

  dis = lax.rsqrt(deg_c[0] + deg_c[1] + 1.0).reshape(N_PAD)[:, None]
  h1w = m1w * dis
  zeros64 = jnp.zeros((N_PAD, 64), jnp.float32)
  zeros32 = jnp.zeros((N_PAD, 32), jnp.float32)

  prop64 = _make_sc_propagate(64)
  prop32 = _make_sc_propagate(32)

  p1 = prop64(h1w.reshape(2 * N_PAD, 64), src_g2, dst_g, zeros64)
  h2w = _tc_combine_matmul(p1, h1w, dis, b1.reshape(1, d1), W2, d1)

  p2 = prop32(h2w.reshape(4 * N_PAD, 32), src_g4, dst_g, zeros32)
  h3w = _tc_combine_matmul(p2, h2w, dis, b2.reshape(1, d2), W3, d2)

  p3 = prop64(h3w.reshape(2 * N_PAD, 64), src_g2, dst_g, zeros64)
  return _tc_final(p3, h3w, dis, b3.reshape(1, d3), d3)[:N_NODES]
